# baseline (speedup 1.0000x reference)
import jax
import jax.numpy as jnp
from jax import lax
from jax.experimental import pallas as pl
from jax.experimental.pallas import tpu as pltpu
from jax.experimental.pallas import tpu_sc as plsc

N = 10000
E = 320000
IN = 128
HID2 = 256
OUT = 64

NC = 2
NS = 16
NP = 10112
RPT = NP // NS
B = 128

BP = 128
EPAD = ((E + NC * NS * 2 * B - 1) // (NC * NS * 2 * B)) * (NC * NS * 2 * B)
EPADP = ((E + NC * NS * 3 * BP - 1) // (NC * NS * 3 * BP)) * (NC * NS * 3 * BP)


def _sc_mesh():
    return plsc.VectorSubcoreMesh(
        core_axis_name="c", subcore_axis_name="s", num_cores=NC, num_subcores=NS
    )


KB = 16


def _deg_body(adj, ones_hbm, zeros_hbm, out, d0, d1, ones_v, acc, i0, i1, ss):
    c = lax.axis_index("c")
    s = lax.axis_index("s")
    ept = EPAD // (NS * NC)
    ebase = (s * NC + c) * ept

    pltpu.sync_copy(zeros_hbm, acc.at[pl.ds(s * RPT, RPT)])
    pltpu.sync_copy(ones_hbm, ones_v)
    plsc.subcore_barrier()

    pltpu.async_copy(adj.at[1, pl.ds(EPAD - B, B)], d1, i1)
    pltpu.make_async_copy(adj.at[1, pl.ds(EPAD - B, B)], d1, i1).wait()
    pltpu.async_copy(ones_v, acc.at[d1], ss, add=True)

    def pair(t, carry):
        a = ebase + t * 2 * B
        b = a + B
        pltpu.async_copy(adj.at[1, pl.ds(a, B)], d0, i0)
        pltpu.make_async_copy(ones_v, acc.at[d1], ss).wait()
        pltpu.async_copy(adj.at[1, pl.ds(b, B)], d1, i1)
        pltpu.make_async_copy(adj.at[1, pl.ds(a, B)], d0, i0).wait()
        pltpu.async_copy(ones_v, acc.at[d0], ss, add=True)
        pltpu.make_async_copy(adj.at[1, pl.ds(b, B)], d1, i1).wait()
        pltpu.make_async_copy(ones_v, acc.at[d0], ss).wait()
        pltpu.async_copy(ones_v, acc.at[d1], ss, add=True)
        return carry

    lax.fori_loop(0, ept // (2 * B), pair, 0)
    pltpu.make_async_copy(ones_v, acc.at[d1], ss).wait()
    plsc.subcore_barrier()
    pltpu.sync_copy(acc.at[pl.ds(s * RPT, RPT)], out.at[c, pl.ds(s * RPT, RPT)])


def _deg_call(adj_pad, ones_b, zeros_c):
    return pl.kernel(
        _deg_body,
        out_type=jax.ShapeDtypeStruct((NC, NP, IN), jnp.float32),
        mesh=_sc_mesh(),
        scratch_types=[
            pltpu.VMEM((B,), jnp.int32),
            pltpu.VMEM((B,), jnp.int32),
            pltpu.VMEM((B, IN), jnp.float32),
            pltpu.VMEM_SHARED((NP, IN), jnp.float32),
            pltpu.SemaphoreType.DMA,
            pltpu.SemaphoreType.DMA,
            pltpu.SemaphoreType.DMA,
        ],
    )(adj_pad, ones_b, zeros_c)


def _prop_body(uh, adj, zeros_hbm, out,
               sr0, sr1, sr2, d0, d1, d2, r0, r1, r2, acc,
               i0, i1, i2, g0, g1, g2, ss):
    c = lax.axis_index("c")
    s = lax.axis_index("s")
    ept = EPADP // (NS * NC)
    ebase = (s * NC + c) * ept

    pltpu.sync_copy(zeros_hbm, acc.at[pl.ds(s * RPT, RPT)])
    plsc.subcore_barrier()

    pltpu.async_copy(adj.at[1, pl.ds(EPADP - BP, BP)], d2, i2)
    pltpu.make_async_copy(adj.at[1, pl.ds(EPADP - BP, BP)], d2, i2).wait()
    pltpu.async_copy(r2, acc.at[d2], ss, add=True)

    def trip(t, carry):
        a = ebase + t * 3 * BP
        b = a + BP
        e = b + BP
        pltpu.async_copy(adj.at[0, pl.ds(a, BP)], sr0, i0)
        pltpu.async_copy(adj.at[1, pl.ds(a, BP)], d0, i0)
        pltpu.async_copy(adj.at[0, pl.ds(b, BP)], sr1, i1)
        pltpu.async_copy(adj.at[1, pl.ds(b, BP)], d1, i1)
        pltpu.make_async_copy(r2, acc.at[d2], ss).wait()
        pltpu.async_copy(adj.at[0, pl.ds(e, BP)], sr2, i2)
        pltpu.async_copy(adj.at[1, pl.ds(e, BP)], d2, i2)
        pltpu.make_async_copy(adj.at[0, pl.ds(a, BP)], sr0, i0).wait()
        pltpu.make_async_copy(adj.at[1, pl.ds(a, BP)], d0, i0).wait()
        pltpu.async_copy(uh.at[sr0], r0, g0)
        pltpu.make_async_copy(adj.at[0, pl.ds(b, BP)], sr1, i1).wait()
        pltpu.make_async_copy(adj.at[1, pl.ds(b, BP)], d1, i1).wait()
        pltpu.async_copy(uh.at[sr1], r1, g1)
        pltpu.make_async_copy(adj.at[0, pl.ds(e, BP)], sr2, i2).wait()
        pltpu.make_async_copy(adj.at[1, pl.ds(e, BP)], d2, i2).wait()
        pltpu.async_copy(uh.at[sr2], r2, g2)
        pltpu.make_async_copy(uh.at[sr0], r0, g0).wait()
        pltpu.async_copy(r0, acc.at[d0], ss, add=True)
        pltpu.make_async_copy(uh.at[sr1], r1, g1).wait()
        pltpu.make_async_copy(r0, acc.at[d0], ss).wait()
        pltpu.async_copy(r1, acc.at[d1], ss, add=True)
        pltpu.make_async_copy(uh.at[sr2], r2, g2).wait()
        pltpu.make_async_copy(r1, acc.at[d1], ss).wait()
        pltpu.async_copy(r2, acc.at[d2], ss, add=True)
        return carry

    lax.fori_loop(0, ept // (3 * BP), trip, 0)
    pltpu.make_async_copy(r2, acc.at[d2], ss).wait()
    plsc.subcore_barrier()
    pltpu.sync_copy(acc.at[pl.ds(s * RPT, RPT)], out.at[c, pl.ds(s * RPT, RPT)])


def _prop_call(uh, adj_pad, zeros_c):
    return pl.kernel(
        _prop_body,
        out_type=jax.ShapeDtypeStruct((NC, NP, IN), jnp.float32),
        mesh=_sc_mesh(),
        scratch_types=[
            pltpu.VMEM((BP,), jnp.int32),
            pltpu.VMEM((BP,), jnp.int32),
            pltpu.VMEM((BP,), jnp.int32),
            pltpu.VMEM((BP,), jnp.int32),
            pltpu.VMEM((BP,), jnp.int32),
            pltpu.VMEM((BP,), jnp.int32),
            pltpu.VMEM((BP, IN), jnp.float32),
            pltpu.VMEM((BP, IN), jnp.float32),
            pltpu.VMEM((BP, IN), jnp.float32),
            pltpu.VMEM_SHARED((NP, IN), jnp.float32),
            pltpu.SemaphoreType.DMA,
            pltpu.SemaphoreType.DMA,
            pltpu.SemaphoreType.DMA,
            pltpu.SemaphoreType.DMA,
            pltpu.SemaphoreType.DMA,
            pltpu.SemaphoreType.DMA,
            pltpu.SemaphoreType.DMA,
        ],
    )(uh, adj_pad, zeros_c)


BR = 1000


def _ux_body(x_ref, degp_ref, ux_ref, dinv_ref):
    deg = degp_ref[0, :, 0:1] + degp_ref[1, :, 0:1] + 1.0
    dinv = lax.rsqrt(deg)
    ux_ref[...] = x_ref[...] * dinv
    dinv_ref[...] = dinv


def _ux_call(x, degp):
    return pl.pallas_call(
        _ux_body,
        grid=(N // BR,),
        in_specs=[
            pl.BlockSpec((BR, IN), lambda i: (i, 0)),
            pl.BlockSpec((NC, BR, IN), lambda i: (0, i, 0)),
        ],
        out_specs=[
            pl.BlockSpec((BR, IN), lambda i: (i, 0)),
            pl.BlockSpec((BR, 1), lambda i: (i, 0)),
        ],
        out_shape=[
            jax.ShapeDtypeStruct((N, IN), jnp.float32),
            jax.ShapeDtypeStruct((N, 1), jnp.float32),
        ],
    )(x, degp)


def _hz2_body(px_ref, ux_ref, dinv_ref, b1_ref, w1_ref, wc_ref, u2_ref):
    dinv = dinv_ref[...]
    ax = (px_ref[0, :, :] + px_ref[1, :, :] + ux_ref[...]) * dinv
    h = jnp.maximum(
        jnp.dot(ax, w1_ref[...], preferred_element_type=jnp.float32)
        + b1_ref[...],
        0.0,
    )
    z2 = jnp.dot(h, wc_ref[...], preferred_element_type=jnp.float32)
    u2_ref[...] = z2 * dinv


def _hz2_call(Px, ux, dinv, b1, W1, Wc):
    return pl.pallas_call(
        _hz2_body,
        grid=(N // BR,),
        in_specs=[
            pl.BlockSpec((NC, BR, IN), lambda i: (0, i, 0)),
            pl.BlockSpec((BR, IN), lambda i: (i, 0)),
            pl.BlockSpec((BR, 1), lambda i: (i, 0)),
            pl.BlockSpec((1, HID2), lambda i: (0, 0)),
            pl.BlockSpec((IN, HID2), lambda i: (0, 0)),
            pl.BlockSpec((HID2, 2 * OUT), lambda i: (0, 0)),
        ],
        out_specs=pl.BlockSpec((BR, 2 * OUT), lambda i: (i, 0)),
        out_shape=jax.ShapeDtypeStruct((N, 2 * OUT), jnp.float32),
    )(Px, ux, dinv, b1, W1, Wc)


def _out_body(p2_ref, u2_ref, dinv_ref, bc_ref, o_ref):
    pre = p2_ref[0, :, :] + p2_ref[1, :, :] + u2_ref[...]
    o_ref[...] = jnp.maximum(pre * dinv_ref[...] + bc_ref[...], 0.0)


def _out_call(P2, u2, dinv, bc):
    return pl.pallas_call(
        _out_body,
        grid=(N // BR,),
        in_specs=[
            pl.BlockSpec((NC, BR, 2 * OUT), lambda i: (0, i, 0)),
            pl.BlockSpec((BR, 2 * OUT), lambda i: (i, 0)),
            pl.BlockSpec((BR, 1), lambda i: (i, 0)),
            pl.BlockSpec((1, 2 * OUT), lambda i: (0, 0)),
        ],
        out_specs=pl.BlockSpec((BR, 2 * OUT), lambda i: (i, 0)),
        out_shape=jax.ShapeDtypeStruct((N, 2 * OUT), jnp.float32),
    )(P2, u2, dinv, bc)


def kernel(x, adj, W1, b1, Wmu, bmu, Ws, bs):
    def pad_to(m):
        pidx = jnp.arange(m - E, dtype=jnp.int32)
        pad_edges = jnp.stack([pidx % N, N + pidx % (NP - N)])
        return jnp.concatenate([adj, pad_edges], axis=1)

    adj_pad = pad_to(EPAD)
    adj_padp = pad_to(EPADP)

    zeros128 = jnp.zeros((RPT, IN), jnp.float32)
    ones_b = jnp.ones((B, IN), jnp.float32)
    Wc = jnp.concatenate([Wmu, Ws], axis=1)
    bc = jnp.concatenate([bmu, bs]).reshape(1, 2 * OUT)
    b1r = b1.reshape(1, HID2)

    degp = _deg_call(adj_pad, ones_b, zeros128)
    ux, dinv = _ux_call(x, degp)
    Px = _prop_call(ux, adj_padp, zeros128)
    u2 = _hz2_call(Px[:, :N, :], ux, dinv, b1r, W1, Wc)
    P2 = _prop_call(u2, adj_padp, zeros128)
    o = _out_call(P2[:, :N, :], u2, dinv, bc)
    return (o[:, :OUT], o[:, OUT:])


# speedup vs baseline: 1.0385x; 1.0385x over previous
import jax
import jax.numpy as jnp
from jax import lax
from jax.experimental import pallas as pl
from jax.experimental.pallas import tpu as pltpu
from jax.experimental.pallas import tpu_sc as plsc

N = 10000
E = 320000
IN = 128
HID2 = 256
OUT = 64

NC = 2
NS = 16
NP = 10112
RPT = NP // NS
B = 128

EPAD = ((E + NC * NS * 2 * B - 1) // (NC * NS * 2 * B)) * (NC * NS * 2 * B)


def _sc_mesh():
    return plsc.VectorSubcoreMesh(
        core_axis_name="c", subcore_axis_name="s", num_cores=NC, num_subcores=NS
    )


KB = 16


def _deg_body(adj, ones_hbm, zeros_hbm, out, d0, d1, ones_v, acc, i0, i1, ss):
    c = lax.axis_index("c")
    s = lax.axis_index("s")
    ept = EPAD // (NS * NC)
    ebase = (s * NC + c) * ept

    pltpu.sync_copy(zeros_hbm, acc.at[pl.ds(s * RPT, RPT)])
    pltpu.sync_copy(ones_hbm, ones_v)
    plsc.subcore_barrier()

    pltpu.async_copy(adj.at[1, pl.ds(EPAD - B, B)], d1, i1)
    pltpu.make_async_copy(adj.at[1, pl.ds(EPAD - B, B)], d1, i1).wait()
    pltpu.async_copy(ones_v, acc.at[d1], ss, add=True)

    def pair(t, carry):
        a = ebase + t * 2 * B
        b = a + B
        pltpu.async_copy(adj.at[1, pl.ds(a, B)], d0, i0)
        pltpu.make_async_copy(ones_v, acc.at[d1], ss).wait()
        pltpu.async_copy(adj.at[1, pl.ds(b, B)], d1, i1)
        pltpu.make_async_copy(adj.at[1, pl.ds(a, B)], d0, i0).wait()
        pltpu.async_copy(ones_v, acc.at[d0], ss, add=True)
        pltpu.make_async_copy(adj.at[1, pl.ds(b, B)], d1, i1).wait()
        pltpu.make_async_copy(ones_v, acc.at[d0], ss).wait()
        pltpu.async_copy(ones_v, acc.at[d1], ss, add=True)
        return carry

    lax.fori_loop(0, ept // (2 * B), pair, 0)
    pltpu.make_async_copy(ones_v, acc.at[d1], ss).wait()
    plsc.subcore_barrier()
    pltpu.sync_copy(acc.at[pl.ds(s * RPT, RPT)], out.at[c, pl.ds(s * RPT, RPT)])


def _deg_call(adj_pad, ones_b, zeros_c):
    return pl.kernel(
        _deg_body,
        out_type=jax.ShapeDtypeStruct((NC, NP, IN), jnp.float32),
        mesh=_sc_mesh(),
        scratch_types=[
            pltpu.VMEM((B,), jnp.int32),
            pltpu.VMEM((B,), jnp.int32),
            pltpu.VMEM((B, IN), jnp.float32),
            pltpu.VMEM_SHARED((NP, IN), jnp.float32),
            pltpu.SemaphoreType.DMA,
            pltpu.SemaphoreType.DMA,
            pltpu.SemaphoreType.DMA,
        ],
    )(adj_pad, ones_b, zeros_c)


def _prop_body(uh, adj, zeros_hbm, out,
               sr0, sr1, d0, d1, r0, r1, acc, i0, i1, g0, g1, ss):
    c = lax.axis_index("c")
    s = lax.axis_index("s")
    ept = EPAD // (NS * NC)
    ebase = (s * NC + c) * ept

    pltpu.sync_copy(zeros_hbm, acc.at[pl.ds(s * RPT, RPT)])
    plsc.subcore_barrier()

    pltpu.async_copy(adj.at[1, pl.ds(EPAD - B, B)], d1, i1)
    pltpu.make_async_copy(adj.at[1, pl.ds(EPAD - B, B)], d1, i1).wait()
    pltpu.async_copy(r1, acc.at[d1], ss, add=True)

    def pair(t, carry):
        a = ebase + t * 2 * B
        b = a + B
        pltpu.async_copy(adj.at[0, pl.ds(a, B)], sr0, i0)
        pltpu.async_copy(adj.at[1, pl.ds(a, B)], d0, i0)
        pltpu.make_async_copy(r1, acc.at[d1], ss).wait()
        pltpu.async_copy(adj.at[0, pl.ds(b, B)], sr1, i1)
        pltpu.async_copy(adj.at[1, pl.ds(b, B)], d1, i1)
        pltpu.make_async_copy(adj.at[0, pl.ds(a, B)], sr0, i0).wait()
        pltpu.make_async_copy(adj.at[1, pl.ds(a, B)], d0, i0).wait()
        pltpu.async_copy(uh.at[sr0], r0, g0)
        pltpu.make_async_copy(adj.at[0, pl.ds(b, B)], sr1, i1).wait()
        pltpu.make_async_copy(adj.at[1, pl.ds(b, B)], d1, i1).wait()
        pltpu.async_copy(uh.at[sr1], r1, g1)
        pltpu.make_async_copy(uh.at[sr0], r0, g0).wait()
        pltpu.async_copy(r0, acc.at[d0], ss, add=True)
        pltpu.make_async_copy(uh.at[sr1], r1, g1).wait()
        pltpu.make_async_copy(r0, acc.at[d0], ss).wait()
        pltpu.async_copy(r1, acc.at[d1], ss, add=True)
        return carry

    lax.fori_loop(0, ept // (2 * B), pair, 0)
    pltpu.make_async_copy(r1, acc.at[d1], ss).wait()
    plsc.subcore_barrier()
    pltpu.sync_copy(acc.at[pl.ds(s * RPT, RPT)], out.at[c, pl.ds(s * RPT, RPT)])


def _prop_call(uh, adj_pad, zeros_c):
    return pl.kernel(
        _prop_body,
        out_type=jax.ShapeDtypeStruct((NC, NP, IN), jnp.float32),
        mesh=_sc_mesh(),
        scratch_types=[
            pltpu.VMEM((B,), jnp.int32),
            pltpu.VMEM((B,), jnp.int32),
            pltpu.VMEM((B,), jnp.int32),
            pltpu.VMEM((B,), jnp.int32),
            pltpu.VMEM((B, IN), jnp.float32),
            pltpu.VMEM((B, IN), jnp.float32),
            pltpu.VMEM_SHARED((NP, IN), jnp.float32),
            pltpu.SemaphoreType.DMA,
            pltpu.SemaphoreType.DMA,
            pltpu.SemaphoreType.DMA,
            pltpu.SemaphoreType.DMA,
            pltpu.SemaphoreType.DMA,
        ],
    )(uh, adj_pad, zeros_c)


BR = 1000


def _ux_body(x_ref, degp_ref, ux_ref, dinv_ref):
    deg = degp_ref[0, :, 0:1] + degp_ref[1, :, 0:1] + 1.0
    dinv = lax.rsqrt(deg)
    ux_ref[...] = x_ref[...] * dinv
    dinv_ref[...] = dinv


def _ux_call(x, degp):
    return pl.pallas_call(
        _ux_body,
        grid=(N // BR,),
        in_specs=[
            pl.BlockSpec((BR, IN), lambda i: (i, 0)),
            pl.BlockSpec((NC, BR, IN), lambda i: (0, i, 0)),
        ],
        out_specs=[
            pl.BlockSpec((BR, IN), lambda i: (i, 0)),
            pl.BlockSpec((BR, 1), lambda i: (i, 0)),
        ],
        out_shape=[
            jax.ShapeDtypeStruct((N, IN), jnp.float32),
            jax.ShapeDtypeStruct((N, 1), jnp.float32),
        ],
    )(x, degp)


def _hz2_body(px_ref, ux_ref, dinv_ref, b1_ref, w1_ref, wc_ref, u2_ref):
    dinv = dinv_ref[...]
    ax = (px_ref[0, :, :] + px_ref[1, :, :] + ux_ref[...]) * dinv
    h = jnp.maximum(
        jnp.dot(ax, w1_ref[...], preferred_element_type=jnp.float32)
        + b1_ref[...],
        0.0,
    )
    z2 = jnp.dot(h, wc_ref[...], preferred_element_type=jnp.float32)
    u2_ref[...] = z2 * dinv


def _hz2_call(Px, ux, dinv, b1, W1, Wc):
    return pl.pallas_call(
        _hz2_body,
        grid=(N // BR,),
        in_specs=[
            pl.BlockSpec((NC, BR, IN), lambda i: (0, i, 0)),
            pl.BlockSpec((BR, IN), lambda i: (i, 0)),
            pl.BlockSpec((BR, 1), lambda i: (i, 0)),
            pl.BlockSpec((1, HID2), lambda i: (0, 0)),
            pl.BlockSpec((IN, HID2), lambda i: (0, 0)),
            pl.BlockSpec((HID2, 2 * OUT), lambda i: (0, 0)),
        ],
        out_specs=pl.BlockSpec((BR, 2 * OUT), lambda i: (i, 0)),
        out_shape=jax.ShapeDtypeStruct((N, 2 * OUT), jnp.float32),
    )(Px, ux, dinv, b1, W1, Wc)


def _out_body(p2_ref, u2_ref, dinv_ref, bc_ref, o_ref):
    pre = p2_ref[0, :, :] + p2_ref[1, :, :] + u2_ref[...]
    o_ref[...] = jnp.maximum(pre * dinv_ref[...] + bc_ref[...], 0.0)


def _out_call(P2, u2, dinv, bc):
    return pl.pallas_call(
        _out_body,
        grid=(N // BR,),
        in_specs=[
            pl.BlockSpec((NC, BR, 2 * OUT), lambda i: (0, i, 0)),
            pl.BlockSpec((BR, 2 * OUT), lambda i: (i, 0)),
            pl.BlockSpec((BR, 1), lambda i: (i, 0)),
            pl.BlockSpec((1, 2 * OUT), lambda i: (0, 0)),
        ],
        out_specs=pl.BlockSpec((BR, 2 * OUT), lambda i: (i, 0)),
        out_shape=jax.ShapeDtypeStruct((N, 2 * OUT), jnp.float32),
    )(P2, u2, dinv, bc)


def kernel(x, adj, W1, b1, Wmu, bmu, Ws, bs):
    pidx = jnp.arange(EPAD - E, dtype=jnp.int32)
    pad_edges = jnp.stack([pidx % N, N + pidx % (NP - N)])
    adj_pad = jnp.concatenate([adj, pad_edges], axis=1)

    zeros128 = jnp.zeros((RPT, IN), jnp.float32)
    ones_b = jnp.ones((B, IN), jnp.float32)
    Wc = jnp.concatenate([Wmu, Ws], axis=1)
    bc = jnp.concatenate([bmu, bs]).reshape(1, 2 * OUT)
    b1r = b1.reshape(1, HID2)

    degp = _deg_call(adj_pad, ones_b, zeros128)
    ux, dinv = _ux_call(x, degp)
    Px = _prop_call(ux, adj_pad, zeros128)
    u2 = _hz2_call(Px[:, :N, :], ux, dinv, b1r, W1, Wc)
    P2 = _prop_call(u2, adj_pad, zeros128)
    o = _out_call(P2[:, :N, :], u2, dinv, bc)
    return (o[:, :OUT], o[:, OUT:])
